# baseline (speedup 1.0000x reference)
import functools
import math

import numpy as np
import jax
import jax.numpy as jnp
from jax import lax
from jax.experimental import pallas as pl
from jax.experimental.pallas import tpu as pltpu
from jax.experimental.pallas import tpu_sc as plsc

_PE_LEN = 2048

try:
    _info = plsc.get_sparse_core_info()
    _NC, _NS, _L = _info.num_cores, _info.num_subcores, _info.num_lanes
except Exception:
    _NC, _NS, _L = 2, 16, 16
_NW = _NC * _NS


def _pos_encoding_np(length, depth):
    half = depth / 2
    positions = np.arange(length)[:, np.newaxis]
    depths = np.arange(half)[np.newaxis, :] / half
    angle_rates = 1 / 10000 ** depths
    angle_rads = positions * angle_rates
    return np.concatenate(
        [np.sin(angle_rads), np.cos(angle_rads)], axis=-1
    ).astype(np.float32)


@functools.lru_cache(maxsize=None)
def _make_sc_gather(V, D, B, seq_len, chunk):
    b_per_w = B // _NW
    n_chunks = b_per_w // chunk
    n_j = D // _L
    scale = float(np.sqrt(np.float32(D)))
    mesh = plsc.VectorSubcoreMesh(core_axis_name="c", subcore_axis_name="s")

    @functools.partial(
        pl.kernel,
        mesh=mesh,
        out_type=jax.ShapeDtypeStruct((B, D), jnp.float32),
        scratch_types=[
            pltpu.VMEM((b_per_w,), jnp.int32),
            pltpu.VMEM((chunk, D), jnp.float32),
            pltpu.VMEM((chunk, D), jnp.float32),
            pltpu.SemaphoreType.DMA,
        ],
    )
    def k(table_hbm, idx_hbm, pos_hbm, out_hbm, idx_v, rows_v, pos_v, sem):
        wid = lax.axis_index("s") * _NC + lax.axis_index("c")
        base = wid * b_per_w
        pltpu.sync_copy(idx_hbm.at[pl.ds(base, b_per_w)], idx_v)
        t_base = base % seq_len
        for c in range(n_chunks):
            off = c * chunk
            gather = pltpu.async_copy(
                table_hbm.at[idx_v.at[pl.ds(off, chunk)]], rows_v, sem
            )
            pltpu.sync_copy(pos_hbm.at[pl.ds(t_base + off, chunk)], pos_v)
            gather.wait()

            def row_body(r, carry):
                for j in range(n_j):
                    sl = pl.ds(j * _L, _L)
                    rows_v[r, sl] = rows_v[r, sl] * scale + pos_v[r, sl]
                return carry

            lax.fori_loop(0, chunk, row_body, 0)
            pltpu.sync_copy(rows_v, out_hbm.at[pl.ds(base + off, chunk)])

    return k


def kernel(x, table):
    bt, seq_len = x.shape
    _, d_model = table.shape
    b = bt * seq_len
    pos = jnp.asarray(_pos_encoding_np(_PE_LEN, d_model)[:seq_len])
    k = _make_sc_gather(table.shape[0], d_model, b, seq_len, 64)
    out = k(table, x.reshape(b), pos)
    return out.reshape(bt, seq_len, d_model)


# speedup vs baseline: 1.1840x; 1.1840x over previous
import functools
import math

import numpy as np
import jax
import jax.numpy as jnp
from jax import lax
from jax.experimental import pallas as pl
from jax.experimental.pallas import tpu as pltpu
from jax.experimental.pallas import tpu_sc as plsc

_PE_LEN = 2048

try:
    _info = plsc.get_sparse_core_info()
    _NC, _NS, _L = _info.num_cores, _info.num_subcores, _info.num_lanes
except Exception:
    _NC, _NS, _L = 2, 16, 16
_NW = _NC * _NS


def _pos_encoding_np(length, depth):
    half = depth / 2
    positions = np.arange(length)[:, np.newaxis]
    depths = np.arange(half)[np.newaxis, :] / half
    angle_rates = 1 / 10000 ** depths
    angle_rads = positions * angle_rates
    return np.concatenate(
        [np.sin(angle_rads), np.cos(angle_rads)], axis=-1
    ).astype(np.float32)


@functools.lru_cache(maxsize=None)
def _make_sc_gather(V, D, B, seq_len, chunk):
    b_per_w = B // _NW
    n_chunks = b_per_w // chunk
    n_j = D // _L
    scale = float(np.sqrt(np.float32(D)))
    mesh = plsc.VectorSubcoreMesh(core_axis_name="c", subcore_axis_name="s")

    @functools.partial(
        pl.kernel,
        mesh=mesh,
        out_type=jax.ShapeDtypeStruct((B, D), jnp.float32),
        scratch_types=[
            pltpu.VMEM((b_per_w,), jnp.int32),
            pltpu.VMEM((chunk, D), jnp.float32),
            pltpu.VMEM((chunk, D), jnp.float32),
            pltpu.VMEM((chunk, D), jnp.float32),
            pltpu.VMEM((chunk, D), jnp.float32),
            pltpu.SemaphoreType.DMA,
            pltpu.SemaphoreType.DMA,
            pltpu.SemaphoreType.DMA,
            pltpu.SemaphoreType.DMA,
        ],
    )
    def k(table_hbm, idx_hbm, pos_hbm, out_hbm, idx_v,
          rows_a, rows_b, pos_a, pos_b, gsem_a, gsem_b, osem_a, osem_b):
        wid = lax.axis_index("s") * _NC + lax.axis_index("c")
        base = wid * b_per_w
        pltpu.sync_copy(idx_hbm.at[pl.ds(base, b_per_w)], idx_v)
        t_base = base % seq_len
        rows = [rows_a, rows_b]
        poss = [pos_a, pos_b]
        gsem = [gsem_a, gsem_b]
        osem = [osem_a, osem_b]
        in_flight = [None, None]
        out_flight = [None, None]

        def fire(c):
            p = c % 2
            in_flight[p] = (
                pltpu.async_copy(
                    table_hbm.at[idx_v.at[pl.ds(c * chunk, chunk)]],
                    rows[p], gsem[p],
                ),
                pltpu.async_copy(
                    pos_hbm.at[pl.ds(t_base + c * chunk, chunk)],
                    poss[p], gsem[p],
                ),
            )

        fire(0)
        for c in range(n_chunks):
            p = c % 2
            if c + 1 < n_chunks:
                q = (c + 1) % 2
                if out_flight[q] is not None:
                    out_flight[q].wait()
                fire(c + 1)
            g, pc = in_flight[p]
            g.wait()
            pc.wait()

            rv, pv = rows[p], poss[p]

            def row_body(r, carry):
                for j in range(n_j):
                    sl = pl.ds(j * _L, _L)
                    rv[r, sl] = rv[r, sl] * scale + pv[r, sl]
                return carry

            lax.fori_loop(0, chunk, row_body, 0)
            out_flight[p] = pltpu.async_copy(
                rows[p], out_hbm.at[pl.ds(base + c * chunk, chunk)], osem[p]
            )
        for f in out_flight:
            if f is not None:
                f.wait()

    return k


def kernel(x, table):
    bt, seq_len = x.shape
    _, d_model = table.shape
    b = bt * seq_len
    pos = jnp.asarray(_pos_encoding_np(_PE_LEN, d_model)[:seq_len])
    k = _make_sc_gather(table.shape[0], d_model, b, seq_len, 32)
    out = k(table, x.reshape(b), pos)
    return out.reshape(bt, seq_len, d_model)
